# baseline (speedup 1.0000x reference)
import functools

import jax
import jax.numpy as jnp
from jax.experimental import pallas as pl
from jax.experimental.pallas import tpu as pltpu

_NPOINTS = [2048, 512, 128, 32]
_NSAMPLE = 32


def _rup(x, m):
    return ((x + m - 1) // m) * m


def _mlp_body(nlayers, final_act, x_ref, *refs):
    out_ref = refs[-1]
    h = x_ref[0]
    for i in range(nlayers):
        w = refs[2 * i][...]
        b = refs[2 * i + 1][...]
        h = jnp.dot(w, h, preferred_element_type=jnp.float32) + b
        if i < nlayers - 1 or final_act:
            h = jnp.maximum(h, 0.0)
    out_ref[0] = h


def _mlp_stack(x, ws, bs, final_act=True, tile_n=512):
    B, Cin, N = x.shape
    Np = _rup(N, 128)
    tn = min(tile_n, Np)
    Np = _rup(Np, tn)
    if Np != N:
        x = jnp.pad(x, ((0, 0), (0, 0), (0, Np - N)))
    nlayers = len(ws)
    Cout = ws[-1].shape[0]
    in_specs = [pl.BlockSpec((1, Cin, tn), lambda b, n: (b, 0, n))]
    args = [x]
    for w, b in zip(ws, bs):
        in_specs.append(pl.BlockSpec(w.shape, lambda b, n: (0, 0)))
        in_specs.append(pl.BlockSpec((w.shape[0], 1), lambda b, n: (0, 0)))
        args.append(w)
        args.append(b.reshape(-1, 1))
    out = pl.pallas_call(
        functools.partial(_mlp_body, nlayers, final_act),
        grid=(B, Np // tn),
        in_specs=in_specs,
        out_specs=pl.BlockSpec((1, Cout, tn), lambda b, n: (b, 0, n)),
        out_shape=jax.ShapeDtypeStruct((B, Cout, Np), jnp.float32),
    )(*args)
    return out[:, :, :N] if Np != N else out


def _knn_body(k, n_real, n_pad, q_ref, rt_ref, idx_ref, val_ref, r2_ref):
    @pl.when(pl.program_id(1) == 0)
    def _():
        rt0 = rt_ref[0]
        r2_ref[...] = jnp.sum(rt0 * rt0, axis=0, keepdims=True)

    q = q_ref[0]
    q2 = (q[:, 0:1] * q[:, 0:1] + q[:, 1:2] * q[:, 1:2]
          + q[:, 2:3] * q[:, 2:3])
    d = q2 + r2_ref[...] - 2.0 * jnp.dot(q, rt_ref[0],
                                         preferred_element_type=jnp.float32)
    colid = jax.lax.broadcasted_iota(jnp.int32, d.shape, 1)
    if n_pad != n_real:
        d = jnp.where(colid >= n_real, jnp.inf, d)
    idxs = []
    vals = []
    for _ in range(k):
        m = jnp.min(d, axis=1, keepdims=True)
        am = jnp.min(jnp.where(d == m, colid, n_pad), axis=1, keepdims=True)
        idxs.append(am)
        vals.append(m)
        d = jnp.where(colid == am, jnp.inf, d)
    idx_ref[0] = jnp.concatenate(idxs, axis=1)
    val_ref[0] = jnp.concatenate(vals, axis=1)


def _knn(query, ref, k, tile_m=8):
    B, M, _ = query.shape
    N = ref.shape[1]
    Np = _rup(N, 128)
    rt = jnp.transpose(ref, (0, 2, 1))
    if Np != N:
        rt = jnp.pad(rt, ((0, 0), (0, 0), (0, Np - N)))
    tm = min(tile_m, M)
    idx, val = pl.pallas_call(
        functools.partial(_knn_body, k, N, Np),
        grid=(B, M // tm),
        in_specs=[
            pl.BlockSpec((1, tm, 3), lambda b, m: (b, m, 0)),
            pl.BlockSpec((1, 3, Np), lambda b, m: (b, 0, 0)),
        ],
        out_specs=[
            pl.BlockSpec((1, tm, k), lambda b, m: (b, m, 0)),
            pl.BlockSpec((1, tm, k), lambda b, m: (b, m, 0)),
        ],
        out_shape=[
            jax.ShapeDtypeStruct((B, M, k), jnp.int32),
            jax.ShapeDtypeStruct((B, M, k), jnp.float32),
        ],
        scratch_shapes=[pltpu.VMEM((1, Np), jnp.float32)],
    )(query, rt)
    return val, idx


def _gather_max(h, idx):
    g = jax.vmap(lambda f, i: jnp.take(f, i, axis=1))(h, idx)
    return jnp.max(g, axis=-1)


def _gather_interp(h, idx, w):
    g = jax.vmap(lambda f, i: jnp.take(f, i, axis=1))(h, idx)
    return jnp.sum(g * w[:, None, :, :], axis=-1)


def kernel(pointcloud, conv0_ws, conv0_bs, sa_ws, sa_bs, fp_ws, fp_bs, cls_ws, cls_bs):
    xyz = pointcloud[..., 0:3]
    feats = jnp.transpose(pointcloud[..., 3:], (0, 2, 1))

    h = _mlp_stack(feats, conv0_ws, conv0_bs)
    _, idx0 = _knn(xyz, xyz, _NSAMPLE)
    f0 = _gather_max(h, idx0)

    l_xyz = [xyz]
    l_feats = [f0]
    for k in range(4):
        q = l_xyz[k][:, :_NPOINTS[k]]
        _, idx = _knn(q, l_xyz[k], _NSAMPLE)
        h = _mlp_stack(l_feats[k], sa_ws[k], sa_bs[k])
        l_xyz.append(q)
        l_feats.append(_gather_max(h, idx))

    for i in range(-1, -5, -1):
        d, idx = _knn(l_xyz[i - 1], l_xyz[i], 3)
        d = jnp.maximum(d, 1e-10)
        w = 1.0 / d
        w = w / jnp.sum(w, -1, keepdims=True)
        interp = _gather_interp(l_feats[i], idx, w)
        x = jnp.concatenate([interp, l_feats[i - 1]], axis=1)
        l_feats[i - 1] = _mlp_stack(x, fp_ws[i], fp_bs[i])

    pred = _mlp_stack(l_feats[0], list(cls_ws), list(cls_bs), final_act=False)
    return jnp.transpose(pred, (0, 2, 1))


# speedup vs baseline: 2.3994x; 2.3994x over previous
import functools

import jax
import jax.numpy as jnp
from jax.experimental import pallas as pl
from jax.experimental.pallas import tpu as pltpu

_NPOINTS = [2048, 512, 128, 32]
_NSAMPLE = 32


def _rup(x, m):
    return ((x + m - 1) // m) * m


def _mlp_body(nlayers, final_act, x_ref, *refs):
    out_ref = refs[-1]
    h = x_ref[0]
    for i in range(nlayers):
        w = refs[2 * i][...]
        b = refs[2 * i + 1][...]
        h = jnp.dot(w, h, preferred_element_type=jnp.float32) + b
        if i < nlayers - 1 or final_act:
            h = jnp.maximum(h, 0.0)
    out_ref[0] = h


def _mlp_stack(x, ws, bs, final_act=True, tile_n=512):
    B, Cin, N = x.shape
    Np = _rup(N, 128)
    tn = min(tile_n, Np)
    Np = _rup(Np, tn)
    if Np != N:
        x = jnp.pad(x, ((0, 0), (0, 0), (0, Np - N)))
    nlayers = len(ws)
    Cout = ws[-1].shape[0]
    in_specs = [pl.BlockSpec((1, Cin, tn), lambda b, n: (b, 0, n))]
    args = [x]
    for w, b in zip(ws, bs):
        in_specs.append(pl.BlockSpec(w.shape, lambda b, n: (0, 0)))
        in_specs.append(pl.BlockSpec((w.shape[0], 1), lambda b, n: (0, 0)))
        args.append(w)
        args.append(b.reshape(-1, 1))
    out = pl.pallas_call(
        functools.partial(_mlp_body, nlayers, final_act),
        grid=(B, Np // tn),
        in_specs=in_specs,
        out_specs=pl.BlockSpec((1, Cout, tn), lambda b, n: (b, 0, n)),
        out_shape=jax.ShapeDtypeStruct((B, Cout, Np), jnp.float32),
    )(*args)
    return out[:, :, :N] if Np != N else out


def _knn_body(k, n_real, n_pad, qt_ref, r_ref, idx_ref, val_ref, s_ref):
    qt = qt_ref[0]
    r = r_ref[0]
    q2 = jnp.sum(qt * qt, axis=0, keepdims=True)
    r2 = jnp.sum(r * r, axis=1, keepdims=True)
    s = r2 - 2.0 * jnp.dot(r, qt, preferred_element_type=jnp.float32)
    if n_pad != n_real:
        rid = jax.lax.broadcasted_iota(jnp.int32, s.shape, 0)
        s = jnp.where(rid >= n_real, jnp.inf, s)
    s_ref[...] = s

    def body(j, carry):
        s = s_ref[...]
        rowid = jax.lax.broadcasted_iota(jnp.int32, s.shape, 0)
        m = jnp.min(s, axis=0, keepdims=True)
        am = jnp.min(jnp.where(s == m, rowid, n_pad), axis=0, keepdims=True)
        idx_ref[0, pl.ds(j, 1)] = am
        val_ref[0, pl.ds(j, 1)] = m + q2
        s_ref[...] = jnp.where(rowid == am, jnp.inf, s)
        return carry

    jax.lax.fori_loop(0, k, body, 0)


def _knn(query, ref, k, tile_m=128):
    B, M, _ = query.shape
    N = ref.shape[1]
    Np = _rup(N, 8)
    if Np != N:
        ref = jnp.pad(ref, ((0, 0), (0, Np - N), (0, 0)))
    qt = jnp.transpose(query, (0, 2, 1))
    Mp = _rup(M, tile_m)
    if Mp != M:
        qt = jnp.pad(qt, ((0, 0), (0, 0), (0, Mp - M)))
    kp = _rup(k, 8)
    idx, val = pl.pallas_call(
        functools.partial(_knn_body, k, N, Np),
        grid=(B, Mp // tile_m),
        in_specs=[
            pl.BlockSpec((1, 3, tile_m), lambda b, m: (b, 0, m)),
            pl.BlockSpec((1, Np, 3), lambda b, m: (b, 0, 0)),
        ],
        out_specs=[
            pl.BlockSpec((1, kp, tile_m), lambda b, m: (b, 0, m)),
            pl.BlockSpec((1, kp, tile_m), lambda b, m: (b, 0, m)),
        ],
        out_shape=[
            jax.ShapeDtypeStruct((B, kp, Mp), jnp.int32),
            jax.ShapeDtypeStruct((B, kp, Mp), jnp.float32),
        ],
        scratch_shapes=[pltpu.VMEM((Np, tile_m), jnp.float32)],
    )(qt, ref)
    idx = jnp.transpose(idx[:, :k, :M], (0, 2, 1))
    val = jnp.transpose(val[:, :k, :M], (0, 2, 1))
    return val, idx


def _gather_max(h, idx):
    g = jax.vmap(lambda f, i: jnp.take(f, i, axis=1))(h, idx)
    return jnp.max(g, axis=-1)


def _gather_interp(h, idx, w):
    g = jax.vmap(lambda f, i: jnp.take(f, i, axis=1))(h, idx)
    return jnp.sum(g * w[:, None, :, :], axis=-1)


def kernel(pointcloud, conv0_ws, conv0_bs, sa_ws, sa_bs, fp_ws, fp_bs, cls_ws, cls_bs):
    xyz = pointcloud[..., 0:3]
    feats = jnp.transpose(pointcloud[..., 3:], (0, 2, 1))

    h = _mlp_stack(feats, conv0_ws, conv0_bs)
    _, idx0 = _knn(xyz, xyz, _NSAMPLE)
    f0 = _gather_max(h, idx0)

    l_xyz = [xyz]
    l_feats = [f0]
    for k in range(4):
        q = l_xyz[k][:, :_NPOINTS[k]]
        if k == 0:
            idx = idx0[:, :_NPOINTS[0]]
        else:
            _, idx = _knn(q, l_xyz[k], _NSAMPLE)
        h = _mlp_stack(l_feats[k], sa_ws[k], sa_bs[k])
        l_xyz.append(q)
        l_feats.append(_gather_max(h, idx))

    for i in range(-1, -5, -1):
        d, idx = _knn(l_xyz[i - 1], l_xyz[i], 3)
        d = jnp.maximum(d, 1e-10)
        w = 1.0 / d
        w = w / jnp.sum(w, -1, keepdims=True)
        interp = _gather_interp(l_feats[i], idx, w)
        x = jnp.concatenate([interp, l_feats[i - 1]], axis=1)
        l_feats[i - 1] = _mlp_stack(x, fp_ws[i], fp_bs[i])

    pred = _mlp_stack(l_feats[0], list(cls_ws), list(cls_bs), final_act=False)
    return jnp.transpose(pred, (0, 2, 1))


# speedup vs baseline: 6.5971x; 2.7495x over previous
import functools

import jax
import jax.numpy as jnp
from jax.experimental import pallas as pl
from jax.experimental.pallas import tpu as pltpu
from jax.experimental.pallas import tpu_sc as plsc

_NPOINTS = [2048, 512, 128, 32]
_NSAMPLE = 32
_NW = 32


def _rup(x, m):
    return ((x + m - 1) // m) * m


def _mlp_body(nlayers, final_act, x_ref, *refs):
    out_ref = refs[-1]
    h = x_ref[0]
    for i in range(nlayers):
        wt = refs[2 * i][...]
        b = refs[2 * i + 1][...]
        h = jnp.dot(h, wt, preferred_element_type=jnp.float32) + b
        if i < nlayers - 1 or final_act:
            h = jnp.maximum(h, 0.0)
    out_ref[0] = h


def _mlp_stack(x, ws, bs, final_act=True, tile_n=512):
    B, N, Cin = x.shape
    tn = min(tile_n, _rup(N, 128))
    Np = _rup(N, tn)
    if Np != N:
        x = jnp.pad(x, ((0, 0), (0, Np - N), (0, 0)))
    nlayers = len(ws)
    Cout = ws[-1].shape[0]
    in_specs = [pl.BlockSpec((1, tn, Cin), lambda b, n: (b, n, 0))]
    args = [x]
    for w, b in zip(ws, bs):
        in_specs.append(pl.BlockSpec((w.shape[1], w.shape[0]), lambda b, n: (0, 0)))
        in_specs.append(pl.BlockSpec((1, w.shape[0]), lambda b, n: (0, 0)))
        args.append(w.T)
        args.append(b.reshape(1, -1))
    out = pl.pallas_call(
        functools.partial(_mlp_body, nlayers, final_act),
        grid=(B, Np // tn),
        in_specs=in_specs,
        out_specs=pl.BlockSpec((1, tn, Cout), lambda b, n: (b, n, 0)),
        out_shape=jax.ShapeDtypeStruct((B, Np, Cout), jnp.float32),
    )(*args)
    return out[:, :N] if Np != N else out


def _knn_body(k, n_real, n_pad, qt_ref, r_ref, idx_ref, val_ref, s_ref):
    qt = qt_ref[0]
    r = r_ref[0]
    q2 = jnp.sum(qt * qt, axis=0, keepdims=True)
    r2 = jnp.sum(r * r, axis=1, keepdims=True)
    s = r2 - 2.0 * jnp.dot(r, qt, preferred_element_type=jnp.float32)
    if n_pad != n_real:
        rid = jax.lax.broadcasted_iota(jnp.int32, s.shape, 0)
        s = jnp.where(rid >= n_real, jnp.inf, s)
    s_ref[...] = s

    def body(j, carry):
        s = s_ref[...]
        rowid = jax.lax.broadcasted_iota(jnp.int32, s.shape, 0)
        m = jnp.min(s, axis=0, keepdims=True)
        am = jnp.min(jnp.where(s == m, rowid, n_pad), axis=0, keepdims=True)
        idx_ref[0, pl.ds(j, 1)] = am
        val_ref[0, pl.ds(j, 1)] = m + q2
        s_ref[...] = jnp.where(rowid == am, jnp.inf, s)
        return carry

    jax.lax.fori_loop(0, k, body, 0)


def _knn(query, ref, k, tile_m=128):
    B, M, _ = query.shape
    N = ref.shape[1]
    Np = _rup(N, 8)
    if Np != N:
        ref = jnp.pad(ref, ((0, 0), (0, Np - N), (0, 0)))
    qt = jnp.transpose(query, (0, 2, 1))
    Mp = _rup(M, tile_m)
    if Mp != M:
        qt = jnp.pad(qt, ((0, 0), (0, 0), (0, Mp - M)))
    kp = _rup(k, 8)
    idx, val = pl.pallas_call(
        functools.partial(_knn_body, k, N, Np),
        grid=(B, Mp // tile_m),
        in_specs=[
            pl.BlockSpec((1, 3, tile_m), lambda b, m: (b, 0, m)),
            pl.BlockSpec((1, Np, 3), lambda b, m: (b, 0, 0)),
        ],
        out_specs=[
            pl.BlockSpec((1, kp, tile_m), lambda b, m: (b, 0, m)),
            pl.BlockSpec((1, kp, tile_m), lambda b, m: (b, 0, m)),
        ],
        out_shape=[
            jax.ShapeDtypeStruct((B, kp, Mp), jnp.int32),
            jax.ShapeDtypeStruct((B, kp, Mp), jnp.float32),
        ],
        scratch_shapes=[pltpu.VMEM((Np, tile_m), jnp.float32)],
    )(qt, ref)
    idx = jnp.transpose(idx[:, :k, :M], (0, 2, 1))
    val = jnp.transpose(val[:, :k, :M], (0, 2, 1))
    return val, idx


def _sc_gather_max(table, flat_idx, K):
    QK = flat_idx.shape[0]
    Q = QK // K
    C = table.shape[1]
    qpw = Q // _NW
    qc = max(1, min(128 // K, 4000 // ((C // 16) * (K + 1)), qpw))
    while qpw % qc:
        qc -= 1
    nch = qpw // qc
    mesh = plsc.VectorSubcoreMesh(core_axis_name="c", subcore_axis_name="s")

    @functools.partial(
        pl.kernel, mesh=mesh,
        out_type=jax.ShapeDtypeStruct((Q, C), jnp.float32),
        scratch_types=[
            pltpu.VMEM((qc * K,), jnp.int32),
            pltpu.VMEM((qc * K, C), jnp.float32),
            pltpu.VMEM((qc, C), jnp.float32),
            pltpu.SemaphoreType.DMA,
        ],
    )
    def gmax(idx_hbm, table_hbm, out_hbm, idx_v, rows_v, out_v, sem):
        wid = jax.lax.axis_index("s") * 2 + jax.lax.axis_index("c")
        base = wid * qpw

        def chunk(ci, carry):
            qbase = base + ci * qc
            pltpu.sync_copy(idx_hbm.at[pl.ds(qbase * K, qc * K)], idx_v)
            pltpu.async_copy(table_hbm.at[idx_v], rows_v, sem).wait()
            for qq in range(qc):
                for cc in range(C // 16):
                    acc = rows_v[qq * K, pl.ds(cc * 16, 16)]
                    for kk in range(1, K):
                        acc = jnp.maximum(
                            acc, rows_v[qq * K + kk, pl.ds(cc * 16, 16)])
                    out_v[qq, pl.ds(cc * 16, 16)] = acc
            pltpu.sync_copy(out_v, out_hbm.at[pl.ds(qbase, qc)])
            return carry

        jax.lax.fori_loop(0, nch, chunk, 0)

    return gmax(flat_idx, table)


def _sa_gather(h, idx):
    B, N, C = h.shape
    M, K = idx.shape[1], idx.shape[2]
    Cp = _rup(C, 128)
    if Cp != C:
        h = jnp.pad(h, ((0, 0), (0, 0), (0, Cp - C)))
    flat = (idx + (jnp.arange(B) * N)[:, None, None]).reshape(B * M * K)
    out = _sc_gather_max(h.reshape(B * N, Cp), flat, K)
    return out.reshape(B, M, Cp)[:, :, :C]


def _sc_interp(table, flat_idx, flat_w):
    Q3 = flat_idx.shape[0]
    Q = Q3 // 3
    C = table.shape[1]
    qpw = Q // _NW
    qc = max(1, min(128 // 3, 4000 // (3 + (C // 16) * 7), qpw))
    while qpw % qc:
        qc -= 1
    nch = qpw // qc
    mesh = plsc.VectorSubcoreMesh(core_axis_name="c", subcore_axis_name="s")

    @functools.partial(
        pl.kernel, mesh=mesh,
        out_type=jax.ShapeDtypeStruct((Q, C), jnp.float32),
        scratch_types=[
            pltpu.VMEM((qc * 3,), jnp.int32),
            pltpu.VMEM((qc * 3, 16), jnp.float32),
            pltpu.VMEM((qc * 3, C), jnp.float32),
            pltpu.VMEM((qc, C), jnp.float32),
            pltpu.SemaphoreType.DMA,
        ],
    )
    def interp(idx_hbm, w_hbm, table_hbm, out_hbm, idx_v, w_v, rows_v, out_v,
               sem):
        wid = jax.lax.axis_index("s") * 2 + jax.lax.axis_index("c")
        base = wid * qpw

        def chunk(ci, carry):
            qbase = base + ci * qc
            pltpu.sync_copy(idx_hbm.at[pl.ds(qbase * 3, qc * 3)], idx_v)
            pltpu.sync_copy(w_hbm.at[pl.ds(qbase * 3, qc * 3)], w_v)
            pltpu.async_copy(table_hbm.at[idx_v], rows_v, sem).wait()
            for qq in range(qc):
                wv = [w_v[qq * 3 + j] for j in range(3)]
                for cc in range(C // 16):
                    acc = rows_v[qq * 3, pl.ds(cc * 16, 16)] * wv[0]
                    acc = acc + rows_v[qq * 3 + 1, pl.ds(cc * 16, 16)] * wv[1]
                    acc = acc + rows_v[qq * 3 + 2, pl.ds(cc * 16, 16)] * wv[2]
                    out_v[qq, pl.ds(cc * 16, 16)] = acc
            pltpu.sync_copy(out_v, out_hbm.at[pl.ds(qbase, qc)])
            return carry

        jax.lax.fori_loop(0, nch, chunk, 0)

    return interp(flat_idx, flat_w, table)


def _fp_interp(kn_feats, idx, w):
    B, Nk, C = kn_feats.shape
    M = idx.shape[1]
    Cp = _rup(C, 128)
    if Cp != C:
        kn_feats = jnp.pad(kn_feats, ((0, 0), (0, 0), (0, Cp - C)))
    flat = (idx + (jnp.arange(B) * Nk)[:, None, None]).reshape(B * M * 3)
    wexp = jnp.broadcast_to(w.reshape(B * M * 3, 1), (B * M * 3, 16))
    out = _sc_interp(kn_feats.reshape(B * Nk, Cp), flat, wexp)
    return out.reshape(B, M, Cp)[:, :, :C]


def kernel(pointcloud, conv0_ws, conv0_bs, sa_ws, sa_bs, fp_ws, fp_bs, cls_ws, cls_bs):
    xyz = pointcloud[..., 0:3]
    feats = pointcloud[..., 3:]

    h = _mlp_stack(feats, conv0_ws, conv0_bs)
    _, idx0 = _knn(xyz, xyz, _NSAMPLE)
    f0 = _sa_gather(h, idx0)

    l_xyz = [xyz]
    l_feats = [f0]
    for k in range(4):
        q = l_xyz[k][:, :_NPOINTS[k]]
        if k == 0:
            idx = idx0[:, :_NPOINTS[0]]
        else:
            _, idx = _knn(q, l_xyz[k], _NSAMPLE)
        h = _mlp_stack(l_feats[k], sa_ws[k], sa_bs[k])
        l_xyz.append(q)
        l_feats.append(_sa_gather(h, idx))

    for i in range(-1, -5, -1):
        d, idx = _knn(l_xyz[i - 1], l_xyz[i], 3)
        d = jnp.maximum(d, 1e-10)
        w = 1.0 / d
        w = w / jnp.sum(w, -1, keepdims=True)
        interp = _fp_interp(l_feats[i], idx, w)
        x = jnp.concatenate([interp, l_feats[i - 1]], axis=-1)
        l_feats[i - 1] = _mlp_stack(x, fp_ws[i], fp_bs[i])

    return _mlp_stack(l_feats[0], list(cls_ws), list(cls_bs), final_act=False)


# speedup vs baseline: 7.0193x; 1.0640x over previous
import functools

import jax
import jax.numpy as jnp
from jax.experimental import pallas as pl
from jax.experimental.pallas import tpu as pltpu
from jax.experimental.pallas import tpu_sc as plsc

_NPOINTS = [2048, 512, 128, 32]
_NSAMPLE = 32
_NW = 32


def _rup(x, m):
    return ((x + m - 1) // m) * m


def _mlp_body(nlayers, final_act, x_ref, *refs):
    out_ref = refs[-1]
    h = x_ref[0]
    for i in range(nlayers):
        wt = refs[2 * i][...]
        b = refs[2 * i + 1][...]
        h = jnp.dot(h, wt, preferred_element_type=jnp.float32) + b
        if i < nlayers - 1 or final_act:
            h = jnp.maximum(h, 0.0)
    out_ref[0] = h


def _mlp_stack(x, ws, bs, final_act=True, tile_n=512):
    B, N, Cin = x.shape
    tn = min(tile_n, _rup(N, 128))
    Np = _rup(N, tn)
    if Np != N:
        x = jnp.pad(x, ((0, 0), (0, Np - N), (0, 0)))
    nlayers = len(ws)
    Cout = ws[-1].shape[0]
    in_specs = [pl.BlockSpec((1, tn, Cin), lambda b, n: (b, n, 0))]
    args = [x]
    for w, b in zip(ws, bs):
        in_specs.append(pl.BlockSpec((w.shape[1], w.shape[0]), lambda b, n: (0, 0)))
        in_specs.append(pl.BlockSpec((1, w.shape[0]), lambda b, n: (0, 0)))
        args.append(w.T)
        args.append(b.reshape(1, -1))
    out = pl.pallas_call(
        functools.partial(_mlp_body, nlayers, final_act),
        grid=(B, Np // tn),
        in_specs=in_specs,
        out_specs=pl.BlockSpec((1, tn, Cout), lambda b, n: (b, n, 0)),
        out_shape=jax.ShapeDtypeStruct((B, Np, Cout), jnp.float32),
    )(*args)
    return out[:, :N] if Np != N else out


def _knn_body(k, n_real, n_pad, qt_ref, r_ref, idx_ref, val_ref, s_ref):
    qt = qt_ref[0]
    r = r_ref[0]
    q2 = jnp.sum(qt * qt, axis=0, keepdims=True)
    r2 = jnp.sum(r * r, axis=1, keepdims=True)
    s = r2 - 2.0 * jnp.dot(r, qt, preferred_element_type=jnp.float32)
    if n_pad != n_real:
        rid = jax.lax.broadcasted_iota(jnp.int32, s.shape, 0)
        s = jnp.where(rid >= n_real, jnp.inf, s)
    s_ref[...] = s

    def body(j, carry):
        s = s_ref[...]
        rowid = jax.lax.broadcasted_iota(jnp.int32, s.shape, 0)
        m = jnp.min(s, axis=0, keepdims=True)
        am = jnp.min(jnp.where(s == m, rowid, n_pad), axis=0, keepdims=True)
        idx_ref[0, pl.ds(j, 1)] = am
        val_ref[0, pl.ds(j, 1)] = m + q2
        s_ref[...] = jnp.where(rowid == am, jnp.inf, s)
        return carry

    jax.lax.fori_loop(0, k, body, 0)


def _knn(query, ref, k, tile_m=128):
    B, M, _ = query.shape
    N = ref.shape[1]
    Np = _rup(N, 8)
    if Np != N:
        ref = jnp.pad(ref, ((0, 0), (0, Np - N), (0, 0)))
    qt = jnp.transpose(query, (0, 2, 1))
    Mp = _rup(M, tile_m)
    if Mp != M:
        qt = jnp.pad(qt, ((0, 0), (0, 0), (0, Mp - M)))
    kp = _rup(k, 8)
    idx, val = pl.pallas_call(
        functools.partial(_knn_body, k, N, Np),
        grid=(B, Mp // tile_m),
        in_specs=[
            pl.BlockSpec((1, 3, tile_m), lambda b, m: (b, 0, m)),
            pl.BlockSpec((1, Np, 3), lambda b, m: (b, 0, 0)),
        ],
        out_specs=[
            pl.BlockSpec((1, kp, tile_m), lambda b, m: (b, 0, m)),
            pl.BlockSpec((1, kp, tile_m), lambda b, m: (b, 0, m)),
        ],
        out_shape=[
            jax.ShapeDtypeStruct((B, kp, Mp), jnp.int32),
            jax.ShapeDtypeStruct((B, kp, Mp), jnp.float32),
        ],
        scratch_shapes=[pltpu.VMEM((Np, tile_m), jnp.float32)],
    )(qt, ref)
    idx = jnp.transpose(idx[:, :k, :M], (0, 2, 1))
    val = jnp.transpose(val[:, :k, :M], (0, 2, 1))
    return val, idx


def _sc_gather_max(table, flat_idx, K, C):
    QK = flat_idx.shape[0]
    Q = QK // K
    Cp = table.shape[1]
    qpw = Q // _NW
    qc = max(1, min(128 // K, 4000 // ((C // 16) * (K + 1)), qpw))
    while qpw % qc:
        qc -= 1
    nch = qpw // qc
    mesh = plsc.VectorSubcoreMesh(core_axis_name="c", subcore_axis_name="s")

    @functools.partial(
        pl.kernel, mesh=mesh,
        out_type=jax.ShapeDtypeStruct((Q, C), jnp.float32),
        scratch_types=[
            pltpu.VMEM((qpw * K,), jnp.int32),
            pltpu.VMEM((qc * K, Cp), jnp.float32),
            pltpu.VMEM((qc * K, Cp), jnp.float32),
            pltpu.VMEM((qc, C), jnp.float32),
            pltpu.SemaphoreType.DMA,
            pltpu.SemaphoreType.DMA,
        ],
    )
    def gmax(idx_hbm, table_hbm, out_hbm, idx_v, rows_v0, rows_v1, out_v,
             sem0, sem1):
        wid = jax.lax.axis_index("s") * 2 + jax.lax.axis_index("c")
        base = wid * qpw
        pltpu.sync_copy(idx_hbm.at[pl.ds(base * K, qpw * K)], idx_v)
        rows = [rows_v0, rows_v1]
        sems = [sem0, sem1]

        def issue(ci, buf):
            pltpu.async_copy(
                table_hbm.at[idx_v.at[pl.ds(ci * (qc * K), qc * K)]],
                rows[buf], sems[buf])

        def compute(ci, buf):
            rv = rows[buf]
            for qq in range(qc):
                for cc in range(C // 16):
                    acc = rv[qq * K, pl.ds(cc * 16, 16)]
                    for kk in range(1, K):
                        acc = jnp.maximum(
                            acc, rv[qq * K + kk, pl.ds(cc * 16, 16)])
                    out_v[qq, pl.ds(cc * 16, 16)] = acc
            pltpu.sync_copy(out_v, out_hbm.at[pl.ds(base + ci * qc, qc)])

        def wait(buf):
            pltpu.make_async_copy(
                table_hbm.at[pl.ds(0, qc * K)], rows[buf], sems[buf]).wait()

        issue(0, 0)
        if nch == 1:
            wait(0)
            compute(0, 0)
        else:
            def pair(p, carry):
                ci0 = 2 * p
                wait(0)
                issue(ci0 + 1, 1)
                compute(ci0, 0)
                wait(1)

                @pl.when(ci0 + 2 < nch)
                def _():
                    issue(ci0 + 2, 0)

                compute(ci0 + 1, 1)
                return carry

            jax.lax.fori_loop(0, nch // 2, pair, 0)

    return gmax(flat_idx, table)


def _sa_gather(h, idx):
    B, N, C = h.shape
    M, K = idx.shape[1], idx.shape[2]
    Cp = _rup(C, 128)
    if Cp != C:
        h = jnp.pad(h, ((0, 0), (0, 0), (0, Cp - C)))
    flat = (idx + (jnp.arange(B) * N)[:, None, None]).reshape(B * M * K)
    out = _sc_gather_max(h.reshape(B * N, Cp), flat, K, C)
    return out.reshape(B, M, C)


def _sc_interp(table, flat_idx, flat_w, C):
    Q3 = flat_idx.shape[0]
    Q = Q3 // 3
    Cp = table.shape[1]
    qpw = Q // _NW
    qc = max(1, min(128 // 3, 4000 // (3 + (C // 16) * 7), qpw))
    while qpw % qc:
        qc -= 1
    nch = qpw // qc
    mesh = plsc.VectorSubcoreMesh(core_axis_name="c", subcore_axis_name="s")

    @functools.partial(
        pl.kernel, mesh=mesh,
        out_type=jax.ShapeDtypeStruct((Q, C), jnp.float32),
        scratch_types=[
            pltpu.VMEM((qc * 3,), jnp.int32),
            pltpu.VMEM((qc * 3, 16), jnp.float32),
            pltpu.VMEM((qc * 3, Cp), jnp.float32),
            pltpu.VMEM((qc, C), jnp.float32),
            pltpu.SemaphoreType.DMA,
        ],
    )
    def interp(idx_hbm, w_hbm, table_hbm, out_hbm, idx_v, w_v, rows_v, out_v,
               sem):
        wid = jax.lax.axis_index("s") * 2 + jax.lax.axis_index("c")
        base = wid * qpw

        def chunk(ci, carry):
            qbase = base + ci * qc
            pltpu.sync_copy(idx_hbm.at[pl.ds(qbase * 3, qc * 3)], idx_v)
            pltpu.sync_copy(w_hbm.at[pl.ds(qbase * 3, qc * 3)], w_v)
            pltpu.async_copy(table_hbm.at[idx_v], rows_v, sem).wait()
            for qq in range(qc):
                wv = [w_v[qq * 3 + j] for j in range(3)]
                for cc in range(C // 16):
                    acc = rows_v[qq * 3, pl.ds(cc * 16, 16)] * wv[0]
                    acc = acc + rows_v[qq * 3 + 1, pl.ds(cc * 16, 16)] * wv[1]
                    acc = acc + rows_v[qq * 3 + 2, pl.ds(cc * 16, 16)] * wv[2]
                    out_v[qq, pl.ds(cc * 16, 16)] = acc
            pltpu.sync_copy(out_v, out_hbm.at[pl.ds(qbase, qc)])
            return carry

        jax.lax.fori_loop(0, nch, chunk, 0)

    return interp(flat_idx, flat_w, table)


def _fp_interp(kn_feats, idx, w):
    B, Nk, C = kn_feats.shape
    M = idx.shape[1]
    Cp = _rup(C, 128)
    if Cp != C:
        kn_feats = jnp.pad(kn_feats, ((0, 0), (0, 0), (0, Cp - C)))
    flat = (idx + (jnp.arange(B) * Nk)[:, None, None]).reshape(B * M * 3)
    wexp = jnp.broadcast_to(w.reshape(B * M * 3, 1), (B * M * 3, 16))
    out = _sc_interp(kn_feats.reshape(B * Nk, Cp), flat, wexp, C)
    return out.reshape(B, M, C)


def kernel(pointcloud, conv0_ws, conv0_bs, sa_ws, sa_bs, fp_ws, fp_bs, cls_ws, cls_bs):
    xyz = pointcloud[..., 0:3]
    feats = pointcloud[..., 3:]

    h = _mlp_stack(feats, conv0_ws, conv0_bs)
    _, idx0 = _knn(xyz, xyz, _NSAMPLE)
    f0 = _sa_gather(h, idx0)

    l_xyz = [xyz]
    l_feats = [f0]
    for k in range(4):
        q = l_xyz[k][:, :_NPOINTS[k]]
        if k == 0:
            idx = idx0[:, :_NPOINTS[0]]
        else:
            _, idx = _knn(q, l_xyz[k], _NSAMPLE)
        h = _mlp_stack(l_feats[k], sa_ws[k], sa_bs[k])
        l_xyz.append(q)
        l_feats.append(_sa_gather(h, idx))

    for i in range(-1, -5, -1):
        d, idx = _knn(l_xyz[i - 1], l_xyz[i], 3)
        d = jnp.maximum(d, 1e-10)
        w = 1.0 / d
        w = w / jnp.sum(w, -1, keepdims=True)
        interp = _fp_interp(l_feats[i], idx, w)
        x = jnp.concatenate([interp, l_feats[i - 1]], axis=-1)
        l_feats[i - 1] = _mlp_stack(x, fp_ws[i], fp_bs[i])

    return _mlp_stack(l_feats[0], list(cls_ws), list(cls_bs), final_act=False)


# speedup vs baseline: 9.9297x; 1.4146x over previous
import functools

import jax
import jax.numpy as jnp
from jax.experimental import pallas as pl
from jax.experimental.pallas import tpu as pltpu
from jax.experimental.pallas import tpu_sc as plsc

_NPOINTS = [2048, 512, 128, 32]
_NSAMPLE = 32
_NW = 32


def _rup(x, m):
    return ((x + m - 1) // m) * m


def _mlp_body(nlayers, final_act, x_ref, *refs):
    out_ref = refs[-1]
    h = x_ref[0]
    for i in range(nlayers):
        wt = refs[2 * i][...]
        b = refs[2 * i + 1][...]
        h = jnp.dot(h, wt, preferred_element_type=jnp.float32) + b
        if i < nlayers - 1 or final_act:
            h = jnp.maximum(h, 0.0)
    out_ref[0] = h


def _mlp_stack(x, ws, bs, final_act=True, tile_n=512):
    B, N, Cin = x.shape
    tn = min(tile_n, _rup(N, 128))
    Np = _rup(N, tn)
    if Np != N:
        x = jnp.pad(x, ((0, 0), (0, Np - N), (0, 0)))
    nlayers = len(ws)
    Cout = ws[-1].shape[0]
    in_specs = [pl.BlockSpec((1, tn, Cin), lambda b, n: (b, n, 0))]
    args = [x]
    for w, b in zip(ws, bs):
        in_specs.append(pl.BlockSpec((w.shape[1], w.shape[0]), lambda b, n: (0, 0)))
        in_specs.append(pl.BlockSpec((1, w.shape[0]), lambda b, n: (0, 0)))
        args.append(w.T)
        args.append(b.reshape(1, -1))
    out = pl.pallas_call(
        functools.partial(_mlp_body, nlayers, final_act),
        grid=(B, Np // tn),
        in_specs=in_specs,
        out_specs=pl.BlockSpec((1, tn, Cout), lambda b, n: (b, n, 0)),
        out_shape=jax.ShapeDtypeStruct((B, Np, Cout), jnp.float32),
    )(*args)
    return out[:, :N] if Np != N else out


def _knn_body(k, n_real, n_pad, qt_ref, r_ref, idx_ref, val_ref, s_ref):
    qt = qt_ref[0]
    r = r_ref[0]
    q2 = jnp.sum(qt * qt, axis=0, keepdims=True)
    r2 = jnp.sum(r * r, axis=1, keepdims=True)
    s = r2 - 2.0 * jnp.dot(r, qt, preferred_element_type=jnp.float32)
    if n_pad != n_real:
        rid = jax.lax.broadcasted_iota(jnp.int32, s.shape, 0)
        s = jnp.where(rid >= n_real, jnp.inf, s)
    s_ref[...] = s

    def body(j, carry):
        s = s_ref[...]
        rowid = jax.lax.broadcasted_iota(jnp.int32, s.shape, 0)
        m = jnp.min(s, axis=0, keepdims=True)
        am = jnp.min(jnp.where(s == m, rowid, n_pad), axis=0, keepdims=True)
        idx_ref[0, pl.ds(j, 1)] = am
        val_ref[0, pl.ds(j, 1)] = m + q2
        s_ref[...] = jnp.where(rowid == am, jnp.inf, s)
        return carry

    jax.lax.fori_loop(0, k, body, 0)


def _knn(query, ref, k, tile_m=256):
    B, M, _ = query.shape
    N = ref.shape[1]
    Np = _rup(N, 8)
    if Np != N:
        ref = jnp.pad(ref, ((0, 0), (0, Np - N), (0, 0)))
    qt = jnp.transpose(query, (0, 2, 1))
    Mp = _rup(M, tile_m)
    if Mp != M:
        qt = jnp.pad(qt, ((0, 0), (0, 0), (0, Mp - M)))
    kp = _rup(k, 8)
    idx, val = pl.pallas_call(
        functools.partial(_knn_body, k, N, Np),
        grid=(B, Mp // tile_m),
        in_specs=[
            pl.BlockSpec((1, 3, tile_m), lambda b, m: (b, 0, m)),
            pl.BlockSpec((1, Np, 3), lambda b, m: (b, 0, 0)),
        ],
        out_specs=[
            pl.BlockSpec((1, kp, tile_m), lambda b, m: (b, 0, m)),
            pl.BlockSpec((1, kp, tile_m), lambda b, m: (b, 0, m)),
        ],
        out_shape=[
            jax.ShapeDtypeStruct((B, kp, Mp), jnp.int32),
            jax.ShapeDtypeStruct((B, kp, Mp), jnp.float32),
        ],
        scratch_shapes=[pltpu.VMEM((Np, tile_m), jnp.float32)],
    )(qt, ref)
    idx = jnp.transpose(idx[:, :k, :M], (0, 2, 1))
    val = jnp.transpose(val[:, :k, :M], (0, 2, 1))
    return val, idx


def _sc_gather_max(table, flat_idx, K, C):
    QK = flat_idx.shape[0]
    Q = QK // K
    Cp = table.shape[1]
    qpw = Q // _NW
    qc = max(1, min(128 // K, 4000 // ((C // 16) * (K + 1)), qpw))
    while qpw % qc:
        qc -= 1
    nch = qpw // qc
    mesh = plsc.VectorSubcoreMesh(core_axis_name="c", subcore_axis_name="s")

    @functools.partial(
        pl.kernel, mesh=mesh,
        out_type=jax.ShapeDtypeStruct((Q, C), jnp.float32),
        scratch_types=[
            pltpu.VMEM((qpw * K,), jnp.int32),
            pltpu.VMEM((qc * K, Cp), jnp.float32),
            pltpu.VMEM((qc * K, Cp), jnp.float32),
            pltpu.VMEM((qc, C), jnp.float32),
            pltpu.SemaphoreType.DMA,
            pltpu.SemaphoreType.DMA,
        ],
    )
    def gmax(idx_hbm, table_hbm, out_hbm, idx_v, rows_v0, rows_v1, out_v,
             sem0, sem1):
        wid = jax.lax.axis_index("s") * 2 + jax.lax.axis_index("c")
        base = wid * qpw
        pltpu.sync_copy(idx_hbm.at[pl.ds(base * K, qpw * K)], idx_v)
        rows = [rows_v0, rows_v1]
        sems = [sem0, sem1]

        def issue(ci, buf):
            pltpu.async_copy(
                table_hbm.at[idx_v.at[pl.ds(ci * (qc * K), qc * K)]],
                rows[buf], sems[buf])

        def compute(ci, buf):
            rv = rows[buf]
            for qq in range(qc):
                for cc in range(C // 16):
                    acc = rv[qq * K, pl.ds(cc * 16, 16)]
                    for kk in range(1, K):
                        acc = jnp.maximum(
                            acc, rv[qq * K + kk, pl.ds(cc * 16, 16)])
                    out_v[qq, pl.ds(cc * 16, 16)] = acc
            pltpu.sync_copy(out_v, out_hbm.at[pl.ds(base + ci * qc, qc)])

        def wait(buf):
            pltpu.make_async_copy(
                table_hbm.at[pl.ds(0, qc * K)], rows[buf], sems[buf]).wait()

        issue(0, 0)
        if nch == 1:
            wait(0)
            compute(0, 0)
        else:
            def pair(p, carry):
                ci0 = 2 * p
                wait(0)
                issue(ci0 + 1, 1)
                compute(ci0, 0)
                wait(1)

                @pl.when(ci0 + 2 < nch)
                def _():
                    issue(ci0 + 2, 0)

                compute(ci0 + 1, 1)
                return carry

            jax.lax.fori_loop(0, nch // 2, pair, 0)

    return gmax(flat_idx, table)


def _sa_gather(h, idx):
    B, N, C = h.shape
    M, K = idx.shape[1], idx.shape[2]
    Cp = _rup(C, 128)
    if Cp != C:
        h = jnp.pad(h, ((0, 0), (0, 0), (0, Cp - C)))
    flat = (idx + (jnp.arange(B) * N)[:, None, None]).reshape(B * M * K)
    out = _sc_gather_max(h.reshape(B * N, Cp), flat, K, C)
    return out.reshape(B, M, C)


def _sc_interp(table, flat_idx, flat_w, C):
    Q3 = flat_idx.shape[0]
    Q = Q3 // 3
    Cp = table.shape[1]
    qpw = Q // _NW
    qc = max(1, min(128 // 3, 4000 // (3 + (C // 16) * 7), qpw))
    while qpw % qc:
        qc -= 1
    nch = qpw // qc
    mesh = plsc.VectorSubcoreMesh(core_axis_name="c", subcore_axis_name="s")

    @functools.partial(
        pl.kernel, mesh=mesh,
        out_type=jax.ShapeDtypeStruct((Q, C), jnp.float32),
        scratch_types=[
            pltpu.VMEM((qc * 3,), jnp.int32),
            pltpu.VMEM((qc * 3, 16), jnp.float32),
            pltpu.VMEM((qc * 3, Cp), jnp.float32),
            pltpu.VMEM((qc, C), jnp.float32),
            pltpu.SemaphoreType.DMA,
        ],
    )
    def interp(idx_hbm, w_hbm, table_hbm, out_hbm, idx_v, w_v, rows_v, out_v,
               sem):
        wid = jax.lax.axis_index("s") * 2 + jax.lax.axis_index("c")
        base = wid * qpw

        def chunk(ci, carry):
            qbase = base + ci * qc
            pltpu.sync_copy(idx_hbm.at[pl.ds(qbase * 3, qc * 3)], idx_v)
            pltpu.sync_copy(w_hbm.at[pl.ds(qbase * 3, qc * 3)], w_v)
            pltpu.async_copy(table_hbm.at[idx_v], rows_v, sem).wait()
            for qq in range(qc):
                wv = [w_v[qq * 3 + j] for j in range(3)]
                for cc in range(C // 16):
                    acc = rows_v[qq * 3, pl.ds(cc * 16, 16)] * wv[0]
                    acc = acc + rows_v[qq * 3 + 1, pl.ds(cc * 16, 16)] * wv[1]
                    acc = acc + rows_v[qq * 3 + 2, pl.ds(cc * 16, 16)] * wv[2]
                    out_v[qq, pl.ds(cc * 16, 16)] = acc
            pltpu.sync_copy(out_v, out_hbm.at[pl.ds(qbase, qc)])
            return carry

        jax.lax.fori_loop(0, nch, chunk, 0)

    return interp(flat_idx, flat_w, table)


def _fp_interp(kn_feats, idx, w):
    B, Nk, C = kn_feats.shape
    M = idx.shape[1]
    Cp = _rup(C, 128)
    if Cp != C:
        kn_feats = jnp.pad(kn_feats, ((0, 0), (0, 0), (0, Cp - C)))
    flat = (idx + (jnp.arange(B) * Nk)[:, None, None]).reshape(B * M * 3)
    wexp = jnp.broadcast_to(w.reshape(B * M * 3, 1), (B * M * 3, 16))
    out = _sc_interp(kn_feats.reshape(B * Nk, Cp), flat, wexp, C)
    return out.reshape(B, M, C)


def kernel(pointcloud, conv0_ws, conv0_bs, sa_ws, sa_bs, fp_ws, fp_bs, cls_ws, cls_bs):
    xyz = pointcloud[..., 0:3]
    feats = pointcloud[..., 3:]

    h = _mlp_stack(feats, conv0_ws, conv0_bs)
    _, idx0 = _knn(xyz, xyz, _NSAMPLE)
    f0 = _sa_gather(h, idx0)

    l_xyz = [xyz]
    l_feats = [f0]
    for k in range(4):
        q = l_xyz[k][:, :_NPOINTS[k]]
        if k == 0:
            idx = idx0[:, :_NPOINTS[0]]
        else:
            _, idx = _knn(q, l_xyz[k], _NSAMPLE)
        h = _mlp_stack(l_feats[k], sa_ws[k], sa_bs[k])
        l_xyz.append(q)
        l_feats.append(_sa_gather(h, idx))

    for i in range(-1, -5, -1):
        d, idx = _knn(l_xyz[i - 1], l_xyz[i], 3)
        d = jnp.maximum(d, 1e-10)
        w = 1.0 / d
        w = w / jnp.sum(w, -1, keepdims=True)
        interp = _fp_interp(l_feats[i], idx, w)
        x = jnp.concatenate([interp, l_feats[i - 1]], axis=-1)
        l_feats[i - 1] = _mlp_stack(x, fp_ws[i], fp_bs[i])

    return _mlp_stack(l_feats[0], list(cls_ws), list(cls_bs), final_act=False)


# speedup vs baseline: 12.2411x; 1.2328x over previous
import functools

import jax
import jax.numpy as jnp
from jax.experimental import pallas as pl
from jax.experimental.pallas import tpu as pltpu
from jax.experimental.pallas import tpu_sc as plsc

_NPOINTS = [2048, 512, 128, 32]
_NSAMPLE = 32
_NW = 32


def _rup(x, m):
    return ((x + m - 1) // m) * m


def _mlp_body(nlayers, final_act, x_ref, *refs):
    out_ref = refs[-1]
    h = x_ref[0]
    for i in range(nlayers):
        wt = refs[2 * i][...]
        b = refs[2 * i + 1][...]
        h = jnp.dot(h, wt, preferred_element_type=jnp.float32) + b
        if i < nlayers - 1 or final_act:
            h = jnp.maximum(h, 0.0)
    out_ref[0] = h


def _mlp_stack(x, ws, bs, final_act=True, tile_n=512):
    B, N, Cin = x.shape
    tn = min(tile_n, _rup(N, 128))
    Np = _rup(N, tn)
    if Np != N:
        x = jnp.pad(x, ((0, 0), (0, Np - N), (0, 0)))
    nlayers = len(ws)
    Cout = ws[-1].shape[0]
    in_specs = [pl.BlockSpec((1, tn, Cin), lambda b, n: (b, n, 0))]
    args = [x]
    for w, b in zip(ws, bs):
        in_specs.append(pl.BlockSpec((w.shape[1], w.shape[0]), lambda b, n: (0, 0)))
        in_specs.append(pl.BlockSpec((1, w.shape[0]), lambda b, n: (0, 0)))
        args.append(w.T)
        args.append(b.reshape(1, -1))
    out = pl.pallas_call(
        functools.partial(_mlp_body, nlayers, final_act),
        grid=(B, Np // tn),
        in_specs=in_specs,
        out_specs=pl.BlockSpec((1, tn, Cout), lambda b, n: (b, n, 0)),
        out_shape=jax.ShapeDtypeStruct((B, Np, Cout), jnp.float32),
    )(*args)
    return out[:, :N] if Np != N else out


def _knn_body(k, n_real, n_pad, qt_ref, r_ref, idx_ref, val_ref, s_ref):
    qt = qt_ref[0]
    r = r_ref[0]
    q2 = jnp.sum(qt * qt, axis=0, keepdims=True)
    r2 = jnp.sum(r * r, axis=1, keepdims=True)
    s = r2 - 2.0 * jnp.dot(r, qt, preferred_element_type=jnp.float32)
    if n_pad != n_real:
        rid = jax.lax.broadcasted_iota(jnp.int32, s.shape, 0)
        s = jnp.where(rid >= n_real, jnp.inf, s)
    s_ref[...] = s

    def body(j, carry):
        s = s_ref[...]
        rowid = jax.lax.broadcasted_iota(jnp.int32, s.shape, 0)
        m = jnp.min(s, axis=0, keepdims=True)
        am = jnp.min(jnp.where(s == m, rowid, n_pad), axis=0, keepdims=True)
        idx_ref[0, pl.ds(j, 1)] = am
        val_ref[0, pl.ds(j, 1)] = m + q2
        s_ref[...] = jnp.where(rowid == am, jnp.inf, s)
        return carry

    jax.lax.fori_loop(0, k, body, 0)


def _knn(query, ref, k, tile_m=512):
    B, M, _ = query.shape
    N = ref.shape[1]
    Np = _rup(N, 8)
    if Np != N:
        ref = jnp.pad(ref, ((0, 0), (0, Np - N), (0, 0)))
    qt = jnp.transpose(query, (0, 2, 1))
    Mp = _rup(M, tile_m)
    if Mp != M:
        qt = jnp.pad(qt, ((0, 0), (0, 0), (0, Mp - M)))
    kp = _rup(k, 8)
    idx, val = pl.pallas_call(
        functools.partial(_knn_body, k, N, Np),
        grid=(B, Mp // tile_m),
        in_specs=[
            pl.BlockSpec((1, 3, tile_m), lambda b, m: (b, 0, m)),
            pl.BlockSpec((1, Np, 3), lambda b, m: (b, 0, 0)),
        ],
        out_specs=[
            pl.BlockSpec((1, kp, tile_m), lambda b, m: (b, 0, m)),
            pl.BlockSpec((1, kp, tile_m), lambda b, m: (b, 0, m)),
        ],
        out_shape=[
            jax.ShapeDtypeStruct((B, kp, Mp), jnp.int32),
            jax.ShapeDtypeStruct((B, kp, Mp), jnp.float32),
        ],
        scratch_shapes=[pltpu.VMEM((Np, tile_m), jnp.float32)],
    )(qt, ref)
    idx = jnp.transpose(idx[:, :k, :M], (0, 2, 1))
    val = jnp.transpose(val[:, :k, :M], (0, 2, 1))
    return val, idx


def _sc_gather_max(table, flat_idx, K, C):
    QK = flat_idx.shape[0]
    Q = QK // K
    Cp = table.shape[1]
    qpw = Q // _NW
    qc = max(1, min(128 // K, 4000 // ((C // 16) * (K + 1)), qpw))
    while qpw % qc:
        qc -= 1
    nch = qpw // qc
    mesh = plsc.VectorSubcoreMesh(core_axis_name="c", subcore_axis_name="s")

    @functools.partial(
        pl.kernel, mesh=mesh,
        out_type=jax.ShapeDtypeStruct((Q, C), jnp.float32),
        scratch_types=[
            pltpu.VMEM((qpw * K,), jnp.int32),
            pltpu.VMEM((qc * K, Cp), jnp.float32),
            pltpu.VMEM((qc * K, Cp), jnp.float32),
            pltpu.VMEM((qc, C), jnp.float32),
            pltpu.SemaphoreType.DMA,
            pltpu.SemaphoreType.DMA,
        ],
    )
    def gmax(idx_hbm, table_hbm, out_hbm, idx_v, rows_v0, rows_v1, out_v,
             sem0, sem1):
        wid = jax.lax.axis_index("s") * 2 + jax.lax.axis_index("c")
        base = wid * qpw
        pltpu.sync_copy(idx_hbm.at[pl.ds(base * K, qpw * K)], idx_v)
        rows = [rows_v0, rows_v1]
        sems = [sem0, sem1]

        def issue(ci, buf):
            pltpu.async_copy(
                table_hbm.at[idx_v.at[pl.ds(ci * (qc * K), qc * K)]],
                rows[buf], sems[buf])

        def compute(ci, buf):
            rv = rows[buf]
            for qq in range(qc):
                for cc in range(C // 16):
                    acc = rv[qq * K, pl.ds(cc * 16, 16)]
                    for kk in range(1, K):
                        acc = jnp.maximum(
                            acc, rv[qq * K + kk, pl.ds(cc * 16, 16)])
                    out_v[qq, pl.ds(cc * 16, 16)] = acc
            pltpu.sync_copy(out_v, out_hbm.at[pl.ds(base + ci * qc, qc)])

        def wait(buf):
            pltpu.make_async_copy(
                table_hbm.at[pl.ds(0, qc * K)], rows[buf], sems[buf]).wait()

        issue(0, 0)
        if nch == 1:
            wait(0)
            compute(0, 0)
        else:
            def pair(p, carry):
                ci0 = 2 * p
                wait(0)
                issue(ci0 + 1, 1)
                compute(ci0, 0)
                wait(1)

                @pl.when(ci0 + 2 < nch)
                def _():
                    issue(ci0 + 2, 0)

                compute(ci0 + 1, 1)
                return carry

            jax.lax.fori_loop(0, nch // 2, pair, 0)

    return gmax(flat_idx, table)


def _sa_gather(h, idx):
    B, N, C = h.shape
    M, K = idx.shape[1], idx.shape[2]
    Cp = _rup(C, 128)
    if Cp != C:
        h = jnp.pad(h, ((0, 0), (0, 0), (0, Cp - C)))
    flat = (idx + (jnp.arange(B) * N)[:, None, None]).reshape(B * M * K)
    out = _sc_gather_max(h.reshape(B * N, Cp), flat, K, C)
    return out.reshape(B, M, C)


def _sc_interp(table, flat_idx, flat_w, C):
    Q3 = flat_idx.shape[0]
    Q = Q3 // 3
    Cp = table.shape[1]
    qpw = Q // _NW
    qc = max(1, min(128 // 3, 4000 // (3 + (C // 16) * 7), qpw))
    while qpw % qc:
        qc -= 1
    nch = qpw // qc
    mesh = plsc.VectorSubcoreMesh(core_axis_name="c", subcore_axis_name="s")

    @functools.partial(
        pl.kernel, mesh=mesh,
        out_type=jax.ShapeDtypeStruct((Q, C), jnp.float32),
        scratch_types=[
            pltpu.VMEM((qc * 3,), jnp.int32),
            pltpu.VMEM((qc * 3, 16), jnp.float32),
            pltpu.VMEM((qc * 3, Cp), jnp.float32),
            pltpu.VMEM((qc, C), jnp.float32),
            pltpu.SemaphoreType.DMA,
        ],
    )
    def interp(idx_hbm, w_hbm, table_hbm, out_hbm, idx_v, w_v, rows_v, out_v,
               sem):
        wid = jax.lax.axis_index("s") * 2 + jax.lax.axis_index("c")
        base = wid * qpw

        def chunk(ci, carry):
            qbase = base + ci * qc
            pltpu.sync_copy(idx_hbm.at[pl.ds(qbase * 3, qc * 3)], idx_v)
            pltpu.sync_copy(w_hbm.at[pl.ds(qbase * 3, qc * 3)], w_v)
            pltpu.async_copy(table_hbm.at[idx_v], rows_v, sem).wait()
            for qq in range(qc):
                wv = [w_v[qq * 3 + j] for j in range(3)]
                for cc in range(C // 16):
                    acc = rows_v[qq * 3, pl.ds(cc * 16, 16)] * wv[0]
                    acc = acc + rows_v[qq * 3 + 1, pl.ds(cc * 16, 16)] * wv[1]
                    acc = acc + rows_v[qq * 3 + 2, pl.ds(cc * 16, 16)] * wv[2]
                    out_v[qq, pl.ds(cc * 16, 16)] = acc
            pltpu.sync_copy(out_v, out_hbm.at[pl.ds(qbase, qc)])
            return carry

        jax.lax.fori_loop(0, nch, chunk, 0)

    return interp(flat_idx, flat_w, table)


def _fp_interp(kn_feats, idx, w):
    B, Nk, C = kn_feats.shape
    M = idx.shape[1]
    Cp = _rup(C, 128)
    if Cp != C:
        kn_feats = jnp.pad(kn_feats, ((0, 0), (0, 0), (0, Cp - C)))
    flat = (idx + (jnp.arange(B) * Nk)[:, None, None]).reshape(B * M * 3)
    wexp = jnp.broadcast_to(w.reshape(B * M * 3, 1), (B * M * 3, 16))
    out = _sc_interp(kn_feats.reshape(B * Nk, Cp), flat, wexp, C)
    return out.reshape(B, M, C)


def kernel(pointcloud, conv0_ws, conv0_bs, sa_ws, sa_bs, fp_ws, fp_bs, cls_ws, cls_bs):
    xyz = pointcloud[..., 0:3]
    feats = pointcloud[..., 3:]

    h = _mlp_stack(feats, conv0_ws, conv0_bs)
    _, idx0 = _knn(xyz, xyz, _NSAMPLE)
    f0 = _sa_gather(h, idx0)

    l_xyz = [xyz]
    l_feats = [f0]
    for k in range(4):
        q = l_xyz[k][:, :_NPOINTS[k]]
        if k == 0:
            idx = idx0[:, :_NPOINTS[0]]
        else:
            _, idx = _knn(q, l_xyz[k], _NSAMPLE)
        h = _mlp_stack(l_feats[k], sa_ws[k], sa_bs[k])
        l_xyz.append(q)
        l_feats.append(_sa_gather(h, idx))

    for i in range(-1, -5, -1):
        d, idx = _knn(l_xyz[i - 1], l_xyz[i], 3)
        d = jnp.maximum(d, 1e-10)
        w = 1.0 / d
        w = w / jnp.sum(w, -1, keepdims=True)
        interp = _fp_interp(l_feats[i], idx, w)
        x = jnp.concatenate([interp, l_feats[i - 1]], axis=-1)
        l_feats[i - 1] = _mlp_stack(x, fp_ws[i], fp_bs[i])

    return _mlp_stack(l_feats[0], list(cls_ws), list(cls_bs), final_act=False)
